# baseline (speedup 1.0000x reference)
import functools

import jax
import jax.numpy as jnp
from jax.experimental import pallas as pl
from jax.experimental.pallas import tpu as pltpu

N_LEVELS = 256
THD_NEG = -128


def _fused_body(qv_ref, e_ref, out_ref, hard_ref, *, chunk):
    @pl.when(pl.program_id(0) == 0)
    def _():
        e = e_ref[:]
        m = jnp.max(e, axis=1, keepdims=True)
        col = jax.lax.broadcasted_iota(jnp.int32, e.shape, 1)
        amax = jnp.min(jnp.where(e == m, col, N_LEVELS), axis=1)
        amax_f = amax.astype(jnp.float32)
        n, d = qv_ref.shape
        idx = qv_ref[:] - THD_NEG
        half_w = 128
        parts = []
        for c in range(d // half_w):
            idxc = jax.lax.slice(idx, (0, c * half_w), (n, (c + 1) * half_w))
            acc = jnp.zeros((n, half_w), jnp.float32)
            for h in range(N_LEVELS // half_w):
                tbl = jnp.broadcast_to(
                    amax_f[None, h * half_w : (h + 1) * half_w], (n, half_w)
                )
                rel = jnp.clip(idxc - h * half_w, 0, half_w - 1)
                g = jnp.take_along_axis(tbl, rel, axis=1)
                acc = jnp.where(idxc // half_w == h, g, acc)
            parts.append(acc)
        hard_ref[:] = jnp.concatenate(parts, axis=1)

    out_ref[:] = jnp.broadcast_to(hard_ref[:][None, :, :], out_ref.shape)


def kernel(quantized_values, encoding_logits):
    n, d = quantized_values.shape
    nl = encoding_logits.shape[0]
    k_per_step = 16
    out = pl.pallas_call(
        functools.partial(_fused_body, chunk=32),
        grid=(nl // k_per_step,),
        in_specs=[
            pl.BlockSpec((n, d), lambda k: (0, 0)),
            pl.BlockSpec((nl, nl), lambda k: (0, 0)),
        ],
        out_specs=pl.BlockSpec((k_per_step, n, d), lambda k: (k, 0, 0)),
        out_shape=jax.ShapeDtypeStruct((nl, n, d), jnp.float32),
        scratch_shapes=[pltpu.VMEM((n, d), jnp.float32)],
    )(quantized_values, encoding_logits)
    return out
